# Initial kernel scaffold; baseline (speedup 1.0000x reference)
import jax
import jax.numpy as jnp
from jax.experimental import pallas as pl


def kernel(species, aev, params):
    raise NotImplementedError("write your pallas kernel here")


# speedup vs baseline: 1.2790x; 1.2790x over previous
import jax
import jax.numpy as jnp
from jax.experimental import pallas as pl

N_SPECIES = 4
ALPHA = 0.1


def _celu(x):
    return jnp.where(x > 0, x, ALPHA * (jnp.exp(x / ALPHA) - 1.0))


def _body(sp_ref, x_ref, w1, b1, w2, b2, w3, b3, w4, b4, out_ref):
    x = x_ref[...]
    sp = sp_ref[...]
    t = x.shape[0]
    acc = jnp.zeros((t, 1), jnp.float32)
    for s in range(N_SPECIES):
        h = _celu(jnp.dot(x, w1[s], preferred_element_type=jnp.float32) + b1[s])
        h = _celu(jnp.dot(h, w2[s], preferred_element_type=jnp.float32) + b2[s])
        h = _celu(jnp.dot(h, w3[s], preferred_element_type=jnp.float32) + b3[s])
        e = jnp.sum(h * w4[s], axis=1, keepdims=True) + b4[s, 0]
        acc = jnp.where(sp == s, e, acc)
    mpt = out_ref.shape[0]
    row = jax.lax.broadcasted_iota(jnp.int32, (mpt, t), 0)
    col = jax.lax.broadcasted_iota(jnp.int32, (mpt, t), 1)
    summat = (col // (t // mpt) == row).astype(jnp.float32)
    out_ref[...] = jnp.dot(summat, acc, preferred_element_type=jnp.float32)


def kernel(species, aev, params):
    b, a = species.shape
    aev_dim = aev.shape[-1]
    n = b * a
    t = 512 if n % 512 == 0 else a
    nt = n // t
    mpt = t // a

    w1 = jnp.stack([params[s][0][0] for s in range(N_SPECIES)])
    b1 = jnp.stack([params[s][0][1] for s in range(N_SPECIES)])
    w2 = jnp.stack([params[s][1][0] for s in range(N_SPECIES)])
    b2 = jnp.stack([params[s][1][1] for s in range(N_SPECIES)])
    w3 = jnp.stack([params[s][2][0] for s in range(N_SPECIES)])
    b3 = jnp.stack([params[s][2][1] for s in range(N_SPECIES)])
    w4 = jnp.stack([params[s][3][0][:, 0] for s in range(N_SPECIES)])
    b4 = jnp.stack([params[s][3][1] for s in range(N_SPECIES)])

    sp3 = species.reshape(n, 1).astype(jnp.int32)
    x2 = aev.reshape(n, aev_dim)

    full = lambda arr: pl.BlockSpec(arr.shape, lambda i: (0,) * arr.ndim)
    out = pl.pallas_call(
        _body,
        grid=(nt,),
        in_specs=[
            pl.BlockSpec((t, 1), lambda i: (i, 0)),
            pl.BlockSpec((t, aev_dim), lambda i: (i, 0)),
            full(w1), full(b1), full(w2), full(b2),
            full(w3), full(b3), full(w4), full(b4),
        ],
        out_specs=pl.BlockSpec((mpt, 1), lambda i: (i, 0)),
        out_shape=jax.ShapeDtypeStruct((b, 1), jnp.float32),
    )(sp3, x2, w1, b1, w2, b2, w3, b3, w4, b4)
    return (species, out.reshape(b))
